# Initial kernel scaffold; baseline (speedup 1.0000x reference)
import jax
import jax.numpy as jnp
from jax.experimental import pallas as pl


def kernel(x, ln_in_g, ln_in_b, ln_r_g, ln_r_b, W_r, b_r, e_ln1_g, e_ln1_b, e_W1, e_b1, e_ln2_g, e_ln2_b, e_W2, e_b2, e_ln3_g, e_ln3_b, e_W3, e_b3):
    raise NotImplementedError("write your pallas kernel here")


# speedup vs baseline: 3.6924x; 3.6924x over previous
import functools

import jax
import jax.numpy as jnp
from jax.experimental import pallas as pl
from jax.experimental.pallas import tpu as pltpu

E = 8
K = 2
D = 768
H = 256
H2 = H // 2
N = 16384
EPS = 1e-5

BT = 512


def _normalize(x):
    m = jnp.mean(x, axis=-1, keepdims=True)
    v = jnp.mean((x - m) ** 2, axis=-1, keepdims=True)
    return (x - m) / jnp.sqrt(v + EPS)


def _gelu(x):
    return 0.5 * x * (1.0 + jax.lax.erf(x * (2.0 ** -0.5)))


def _moe_block(
    x_ref, g_in_ref, b_in_ref, g_r_ref, b_r_ref, W_r_ref, br_ref,
    ln1g_ref, ln1b_ref, W1_ref, b1_ref, ln2g_ref, ln2b_ref,
    W2_ref, b2_ref, ln3g_ref, ln3b_ref, W3_ref, b3_ref,
    out_ref, logits_ref, frac_ref, prob_ref,
):
    i = pl.program_id(0)
    nb = pl.num_programs(0)

    x = x_ref[...]
    xn = _normalize(x) * g_in_ref[...] + b_in_ref[...]
    z = _normalize(xn)

    xr = z * g_r_ref[...] + b_r_ref[...]
    logits = jnp.dot(xr, W_r_ref[...], preferred_element_type=jnp.float32)
    logits = logits + br_ref[...]
    logits_ref[...] = logits

    idx = jax.lax.broadcasted_iota(jnp.int32, (BT, E), 1)
    m1 = jnp.max(logits, axis=1, keepdims=True)
    i1 = jnp.min(jnp.where(logits == m1, idx, E), axis=1, keepdims=True)
    rest = jnp.where(idx == i1, -jnp.inf, logits)
    m2 = jnp.max(rest, axis=1, keepdims=True)
    i2 = jnp.min(jnp.where(rest == m2, idx, E), axis=1, keepdims=True)
    t = jnp.exp(m2 - m1)
    w1 = 1.0 / (1.0 + t)
    w2 = t / (1.0 + t)
    sparse_w = (jnp.where(idx == i1, w1, 0.0)
                + jnp.where(idx == i2, w2, 0.0))
    routed = ((idx == i1) | ((idx == i2) & (w2 > 0.0))).astype(jnp.float32)

    probs = jnp.exp(logits - m1)
    probs = probs / jnp.sum(probs, axis=1, keepdims=True)

    @pl.when(i == 0)
    def _():
        frac_ref[...] = jnp.zeros_like(frac_ref)
        prob_ref[...] = jnp.zeros_like(prob_ref)

    frac_ref[...] += jnp.sum(routed, axis=0, keepdims=True)
    prob_ref[...] += jnp.sum(probs, axis=0, keepdims=True)

    @pl.when(i == nb - 1)
    def _():
        frac_ref[...] *= 1.0 / N
        prob_ref[...] *= 1.0 / N

    ys = []
    for e in range(E):
        h = z * ln1g_ref[e][None, :] + ln1b_ref[e][None, :]
        h = _gelu(jnp.dot(h, W1_ref[e], preferred_element_type=jnp.float32)
                  + b1_ref[e][None, :])
        h = _normalize(h) * ln2g_ref[e][None, :] + ln2b_ref[e][None, :]
        h = _gelu(jnp.dot(h, W2_ref[e], preferred_element_type=jnp.float32)
                  + b2_ref[e][None, :])
        h = _normalize(h) * ln3g_ref[e][None, :] + ln3b_ref[e][None, :]
        ys.append(jnp.sum(h * W3_ref[e][None, :], axis=-1, keepdims=True))
    outs = jnp.concatenate(ys, axis=1)
    final = jnp.sum((outs + b3_ref[...]) * sparse_w, axis=1, keepdims=True)
    out_ref[...] = final


def kernel(x, ln_in_g, ln_in_b, ln_r_g, ln_r_b, W_r, b_r,
           e_ln1_g, e_ln1_b, e_W1, e_b1, e_ln2_g, e_ln2_b,
           e_W2, e_b2, e_ln3_g, e_ln3_b, e_W3, e_b3):
    nb = N // BT

    def rep(shape):
        return pl.BlockSpec(shape, lambda i: (0,) * len(shape))

    out, logits, frac, prob = pl.pallas_call(
        _moe_block,
        grid=(nb,),
        in_specs=[
            pl.BlockSpec((BT, D), lambda i: (i, 0)),
            rep((1, D)), rep((1, D)), rep((1, D)), rep((1, D)),
            rep((D, E)), rep((1, E)),
            rep((E, D)), rep((E, D)), rep((E, D, H)), rep((E, H)),
            rep((E, H)), rep((E, H)), rep((E, H, H2)), rep((E, H2)),
            rep((E, H2)), rep((E, H2)), rep((E, H2)), rep((1, E)),
        ],
        out_specs=[
            pl.BlockSpec((BT, 1), lambda i: (i, 0)),
            pl.BlockSpec((BT, E), lambda i: (i, 0)),
            pl.BlockSpec((1, E), lambda i: (0, 0)),
            pl.BlockSpec((1, E), lambda i: (0, 0)),
        ],
        out_shape=[
            jax.ShapeDtypeStruct((N, 1), jnp.float32),
            jax.ShapeDtypeStruct((N, E), jnp.float32),
            jax.ShapeDtypeStruct((1, E), jnp.float32),
            jax.ShapeDtypeStruct((1, E), jnp.float32),
        ],
        compiler_params=pltpu.CompilerParams(
            dimension_semantics=("arbitrary",),
        ),
    )(
        x,
        ln_in_g.reshape(1, D), ln_in_b.reshape(1, D),
        ln_r_g.reshape(1, D), ln_r_b.reshape(1, D),
        W_r, b_r.reshape(1, E),
        e_ln1_g, e_ln1_b, e_W1, e_b1,
        e_ln2_g, e_ln2_b, e_W2, e_b2,
        e_ln3_g, e_ln3_b, e_W3.reshape(E, H2), e_b3.reshape(1, E),
    )
    return (out, frac.reshape(E), prob.reshape(E), logits)
